# Initial kernel scaffold; baseline (speedup 1.0000x reference)
import jax
import jax.numpy as jnp
from jax.experimental import pallas as pl


def kernel(x, edge_index, edge_attr, batch, pri_num, W, We):
    raise NotImplementedError("write your pallas kernel here")


# speedup vs baseline: 2.2992x; 2.2992x over previous
import functools

import jax
import jax.numpy as jnp
from jax import lax
from jax.experimental import pallas as pl
from jax.experimental.pallas import tpu as pltpu
from jax.experimental.pallas import tpu_sc as plsc

N_NODES = 10000
N_EDGES = 320000
D_FEAT = 128
D_EDGE = 16
N_GRAPHS = 32

NC = 2
NS = 16
NW = NC * NS
CHUNK = 128
CHUNKS_PER_W = 79
E_PAD = NW * CHUNKS_PER_W * CHUNK
R_PAD = 10240
ROWS_PER_TILE = R_PAD // NS


@functools.cache
def _make_sc_x_kernel():
    mesh = plsc.VectorSubcoreMesh(core_axis_name="c", subcore_axis_name="s")

    @functools.partial(
        pl.kernel,
        out_type=jax.ShapeDtypeStruct((NC, R_PAD, D_FEAT), jnp.float32),
        mesh=mesh,
        scratch_types=[
            pltpu.VMEM((CHUNK,), jnp.int32),
            pltpu.VMEM((CHUNK,), jnp.int32),
            pltpu.VMEM((CHUNK, D_FEAT), jnp.float32),
            pltpu.VMEM((16, D_FEAT), jnp.float32),
            pltpu.VMEM_SHARED((R_PAD, D_FEAT), jnp.float32),
            pltpu.SemaphoreType.DMA,
            pltpu.SemaphoreType.DMA,
        ],
    )
    def sc_kernel(x_hbm, src_hbm, dst_hbm, ep_hbm, agg_hbm,
                  idx_src, idx_dst, rows, zbuf, agg_sh, sem_ld, sem_st):
        c = lax.axis_index("c")
        s = lax.axis_index("s")
        wid = c * NS + s
        row0 = s * ROWS_PER_TILE

        zeros16 = jnp.zeros((16,), jnp.float32)
        for r in range(16):
            for j in range(8):
                zbuf[r, pl.ds(j * 16, 16)] = zeros16

        @pl.loop(0, ROWS_PER_TILE // 16)
        def zero_step(i):
            pltpu.sync_copy(zbuf, agg_sh.at[pl.ds(row0 + i * 16, 16)])

        plsc.subcore_barrier()

        ebase = wid * (CHUNKS_PER_W * CHUNK)

        @pl.loop(0, CHUNKS_PER_W)
        def edge_step(i):
            base = ebase + i * CHUNK
            d1 = pltpu.async_copy(src_hbm.at[pl.ds(base, CHUNK)], idx_src,
                                  sem_ld)
            d2 = pltpu.async_copy(dst_hbm.at[pl.ds(base, CHUNK)], idx_dst,
                                  sem_ld)
            d1.wait()
            d2.wait()
            pltpu.async_copy(x_hbm.at[idx_src], rows, sem_ld).wait()
            pltpu.async_copy(rows, agg_sh.at[idx_dst], sem_st, add=True).wait()

        @pl.loop(0, CHUNKS_PER_W)
        def eproj_step(i):
            base = ebase + i * CHUNK
            d1 = pltpu.async_copy(dst_hbm.at[pl.ds(base, CHUNK)], idx_dst,
                                  sem_ld)
            d2 = pltpu.async_copy(ep_hbm.at[pl.ds(base, CHUNK)], rows,
                                  sem_ld)
            d1.wait()
            d2.wait()
            pltpu.async_copy(rows, agg_sh.at[idx_dst], sem_st, add=True).wait()

        plsc.subcore_barrier()

        @pl.loop(0, ROWS_PER_TILE // CHUNK)
        def out_step(i):
            r = row0 + i * CHUNK
            pltpu.async_copy(agg_sh.at[pl.ds(r, CHUNK)], rows, sem_ld).wait()
            pltpu.async_copy(rows, agg_hbm.at[c, pl.ds(r, CHUNK)],
                             sem_st).wait()

    return sc_kernel


def _tc_eproj_body(ea_ref, we_ref, ep_ref):
    hi = jax.lax.Precision.HIGHEST
    ep_ref[...] = lax.dot_general(
        ea_ref[...], we_ref[...], (((1,), (0,)), ((), ())), precision=hi)


def _tc_eproj(ea_p, We):
    blk = 4096
    grid = E_PAD // blk
    return pl.pallas_call(
        _tc_eproj_body,
        grid=(grid,),
        in_specs=[pl.BlockSpec((blk, D_EDGE), lambda i: (i, 0)),
                  pl.BlockSpec((D_EDGE, D_FEAT), lambda i: (0, 0))],
        out_specs=pl.BlockSpec((blk, D_FEAT), lambda i: (i, 0)),
        out_shape=jax.ShapeDtypeStruct((E_PAD, D_FEAT), jnp.float32),
    )(ea_p, We)


def _tc_gnn_body(x_ref, a0_ref, a1_ref, w_ref, nr_ref):
    hi = jax.lax.Precision.HIGHEST
    h = x_ref[...] + a0_ref[...] + a1_ref[...]
    nr_ref[...] = jnp.maximum(
        lax.dot_general(h, w_ref[...], (((1,), (0,)), ((), ())), precision=hi),
        0.0)


def _tc_pool_body(nr_ref, b_ref, p_ref, out_ref):
    f32 = jnp.float32
    hi = jax.lax.Precision.HIGHEST
    node_rep = nr_ref[...]
    b = b_ref[...]
    gids = lax.broadcasted_iota(jnp.int32, (N_NODES, N_GRAPHS), 1)
    onehot = (b == gids).astype(f32)
    starts = jnp.sum((b < gids).astype(f32), axis=0, keepdims=True)
    starts_n = jnp.sum(onehot * starts, axis=1, keepdims=True)
    pri_n = jnp.sum(onehot * p_ref[0:1, :], axis=1, keepdims=True)
    pos = lax.broadcasted_iota(jnp.int32, (N_NODES, 1), 0).astype(f32) - starts_n
    cmask = (pos >= pri_n).astype(f32)
    moh = onehot * cmask
    cond_cnt = jnp.sum(moh, axis=0, keepdims=True)
    cond_sum = lax.dot_general(
        moh, node_rep, (((0,), (0,)), ((), ())), precision=hi)
    oh_scaled = onehot / jnp.maximum(cond_cnt, 1.0)
    pooled = lax.dot_general(
        oh_scaled, cond_sum, (((1,), (0,)), ((), ())), precision=hi)
    out_ref[:, 0:D_FEAT] = node_rep
    out_ref[:, D_FEAT:2 * D_FEAT] = pooled


def _tc_call(x, a0, a1, W, bcol, p8):
    node_rep = pl.pallas_call(
        _tc_gnn_body,
        out_shape=jax.ShapeDtypeStruct((N_NODES, D_FEAT), jnp.float32),
    )(x, a0, a1, W)
    return pl.pallas_call(
        _tc_pool_body,
        out_shape=jax.ShapeDtypeStruct((N_NODES, 2 * D_FEAT), jnp.float32),
    )(node_rep, bcol, p8)


def kernel(x, edge_index, edge_attr, batch, pri_num, W, We):
    pad = E_PAD - N_EDGES
    src_p = jnp.concatenate([edge_index[0], jnp.zeros((pad,), jnp.int32)])
    dst_p = jnp.concatenate(
        [edge_index[1], jnp.full((pad,), N_NODES, jnp.int32)])
    ea_p = jnp.concatenate(
        [edge_attr, jnp.zeros((pad, D_EDGE), jnp.float32)], axis=0)
    eproj = _tc_eproj(ea_p, We)
    aggx = _make_sc_x_kernel()(x, src_p, dst_p, eproj)
    bcol = batch.reshape(N_NODES, 1)
    p8 = jnp.broadcast_to(pri_num.astype(jnp.float32)[None, :], (8, N_GRAPHS))
    return _tc_call(x, aggx[0, :N_NODES], aggx[1, :N_NODES], W, bcol, p8)
